# baseline (speedup 1.0000x reference)
import jax
import jax.numpy as jnp
from jax import lax
from jax.experimental import pallas as pl
from jax.experimental.pallas import tpu as pltpu
from jax.experimental.pallas import tpu_sc as plsc

N = 10000
E = 320000
D = 128

NC, NS = 2, 16
NW = NC * NS
RPT = 640
NPAD = NS * RPT
CH = 128
NCHUNK = 80
NBUF = 4
EPT = NCHUNK * CH
EPAD = NW * EPT

BLK = 640
NBLK = NPAD // BLK

_mesh = lambda: plsc.VectorSubcoreMesh(
    core_axis_name="c", subcore_axis_name="s", num_cores=NC, num_subcores=NS)


def _deg_body(dst_hbm, out_hbm, didx_all, ones_v, zero_v, acc_sh):
    c = lax.axis_index("c")
    s = lax.axis_index("s")
    tid = c * NS + s
    for j in range(CH // 16):
        ones_v[pl.ds(16 * j, 16)] = jnp.ones((16,), jnp.float32)
    for j in range(RPT // 16):
        zero_v[pl.ds(16 * j, 16)] = jnp.zeros((16,), jnp.float32)
    pltpu.sync_copy(dst_hbm.at[pl.ds(tid * NCHUNK, NCHUNK)], didx_all)
    pltpu.sync_copy(zero_v, acc_sh.at[pl.ds(s * RPT, RPT)])
    plsc.subcore_barrier()

    def body(i, carry):
        pltpu.sync_copy(ones_v, acc_sh.at[didx_all.at[i]], add=True)
        return carry

    lax.fori_loop(0, NCHUNK, body, 0)
    plsc.subcore_barrier()
    pltpu.sync_copy(acc_sh.at[pl.ds(s * RPT, RPT)],
                    out_hbm.at[c, pl.ds(s * RPT, RPT)])


def _sc_degree(dst_pad):
    k = pl.kernel(
        _deg_body,
        out_type=jax.ShapeDtypeStruct((NC, NPAD), jnp.float32),
        mesh=_mesh(),
        scratch_types=[
            pltpu.VMEM((NCHUNK, CH), jnp.int32),
            pltpu.VMEM((CH,), jnp.float32),
            pltpu.VMEM((RPT,), jnp.float32),
            pltpu.VMEM_SHARED((NPAD,), jnp.float32),
        ],
    )
    return k(dst_pad)


def _scat_body(ei_hbm, g_hbm, out_hbm, ibuf, rows, acc_sh, sem):
    c = lax.axis_index("c")
    s = lax.axis_index("s")
    tid = c * NS + s

    def zbody(r, carry):
        for j in range(D // 16):
            rows[r, pl.ds(16 * j, 16)] = jnp.zeros((16,), jnp.float32)
        return carry

    lax.fori_loop(0, CH, zbody, 0)
    for k in range(RPT // CH):
        pltpu.sync_copy(rows, acc_sh.at[pl.ds(s * RPT + k * CH, CH)])
    plsc.subcore_barrier()
    base = tid * NCHUNK

    def ebody(i, carry):
        pltpu.sync_copy(ei_hbm.at[base + i], ibuf)
        pltpu.async_copy(g_hbm.at[ibuf.at[0]], rows, sem).wait()
        pltpu.sync_copy(rows, acc_sh.at[ibuf.at[1]], add=True)
        return carry

    lax.fori_loop(0, NCHUNK, ebody, 0)
    plsc.subcore_barrier()
    pltpu.sync_copy(acc_sh.at[pl.ds(s * RPT, RPT)],
                    out_hbm.at[c, pl.ds(s * RPT, RPT)])


def _sc_scatter(ei_pad, g):
    k = pl.kernel(
        _scat_body,
        out_type=jax.ShapeDtypeStruct((NC, NPAD, D), jnp.float32),
        mesh=_mesh(),
        scratch_types=[
            pltpu.VMEM((2, CH), jnp.int32),
            pltpu.VMEM((CH, D), jnp.float32),
            pltpu.VMEM_SHARED((NPAD, D), jnp.float32),
            pltpu.SemaphoreType.DMA,
        ],
    )
    return k(ei_pad, g)


def _mm1_body(x_ref, w_ref, deg_ref, g_ref, dis_ref):
    i = pl.program_id(0)
    deg = deg_ref[0] + deg_ref[1] + 1.0
    row = i * BLK + lax.broadcasted_iota(jnp.int32, (BLK, 1), 0)
    dis = jnp.where(row < N, lax.rsqrt(deg), 0.0)
    dis_ref[...] = dis
    t = jnp.dot(x_ref[...], w_ref[...], preferred_element_type=jnp.float32)
    g_ref[...] = t * dis


def _tc_mm1(x_pad, w, deg_col):
    return pl.pallas_call(
        _mm1_body,
        grid=(NBLK,),
        in_specs=[
            pl.BlockSpec((BLK, D), lambda i: (i, 0)),
            pl.BlockSpec((D, D), lambda i: (0, 0)),
            pl.BlockSpec((NC, BLK, 1), lambda i: (0, i, 0)),
        ],
        out_specs=[
            pl.BlockSpec((BLK, D), lambda i: (i, 0)),
            pl.BlockSpec((BLK, 1), lambda i: (i, 0)),
        ],
        out_shape=[
            jax.ShapeDtypeStruct((NPAD, D), jnp.float32),
            jax.ShapeDtypeStruct((NPAD, 1), jnp.float32),
        ],
    )(x_pad, w, deg_col)


def _ep_body(s_ref, g_ref, dis_ref, b_ref, w_ref, out_ref):
    dis = dis_ref[...]
    h = dis * (s_ref[0] + s_ref[1] + g_ref[...]) + b_ref[...]
    h = jnp.maximum(h, 0.0)
    out_ref[...] = jnp.dot(h, w_ref[...], preferred_element_type=jnp.float32) * dis


def _tc_epilogue(scat, g, dis_col, b, w):
    return pl.pallas_call(
        _ep_body,
        grid=(NBLK,),
        in_specs=[
            pl.BlockSpec((NC, BLK, D), lambda i: (0, i, 0)),
            pl.BlockSpec((BLK, D), lambda i: (i, 0)),
            pl.BlockSpec((BLK, 1), lambda i: (i, 0)),
            pl.BlockSpec((1, D), lambda i: (0, 0)),
            pl.BlockSpec((D, D), lambda i: (0, 0)),
        ],
        out_specs=pl.BlockSpec((BLK, D), lambda i: (i, 0)),
        out_shape=jax.ShapeDtypeStruct((NPAD, D), jnp.float32),
    )(scat, g, dis_col, b, w)


def _fin_body(s_ref, g_ref, dis_ref, b_ref, out_ref):
    out_ref[...] = (dis_ref[...] * (s_ref[0] + s_ref[1] + g_ref[...])
                    + b_ref[...])


def _tc_final(scat, g, dis_col, b):
    return pl.pallas_call(
        _fin_body,
        grid=(NBLK,),
        in_specs=[
            pl.BlockSpec((NC, BLK, D), lambda i: (0, i, 0)),
            pl.BlockSpec((BLK, D), lambda i: (i, 0)),
            pl.BlockSpec((BLK, 1), lambda i: (i, 0)),
            pl.BlockSpec((1, D), lambda i: (0, 0)),
        ],
        out_specs=pl.BlockSpec((BLK, D), lambda i: (i, 0)),
        out_shape=jax.ShapeDtypeStruct((NPAD, D), jnp.float32),
    )(scat, g, dis_col, b)


def kernel(x, edge_index, W1, b1, W2, b2, W3, b3):
    src = edge_index[0].astype(jnp.int32)
    dst = edge_index[1].astype(jnp.int32)
    npad_e = EPAD - E
    src_pad = jnp.concatenate(
        [src, jnp.full((npad_e,), N, jnp.int32)]).reshape(EPAD // CH, CH)
    dst_pad = jnp.concatenate(
        [dst, jnp.full((npad_e,), N + 8, jnp.int32)]).reshape(EPAD // CH, CH)
    ei_pad = jnp.stack([src_pad, dst_pad], axis=1)
    x_pad = jnp.concatenate([x, jnp.zeros((NPAD - N, D), jnp.float32)])
    b1c = b1.reshape(1, D)
    b2c = b2.reshape(1, D)
    b3c = b3.reshape(1, D)

    deg = _sc_degree(dst_pad)
    deg_col = deg.reshape(NC, NPAD, 1)
    g1, dis_col = _tc_mm1(x_pad, W1, deg_col)
    s1 = _sc_scatter(ei_pad, g1)
    g2 = _tc_epilogue(s1, g1, dis_col, b1c, W2)
    s2 = _sc_scatter(ei_pad, g2)
    g3 = _tc_epilogue(s2, g2, dis_col, b2c, W3)
    s3 = _sc_scatter(ei_pad, g3)
    out = _tc_final(s3, g3, dis_col, b3c)
    return out[:N]


# speedup vs baseline: 1.2002x; 1.2002x over previous
import jax
import jax.numpy as jnp
from jax import lax
from jax.experimental import pallas as pl
from jax.experimental.pallas import tpu as pltpu
from jax.experimental.pallas import tpu_sc as plsc

N = 10000
E = 320000
D = 128

NC, NS = 2, 16
NW = NC * NS
RPT = 640
NPAD = NS * RPT
CH = 128
NCHUNK = 80
EPT = NCHUNK * CH
EPAD = NW * EPT

BLK = 640
NBLK = NPAD // BLK

_mesh = lambda: plsc.VectorSubcoreMesh(
    core_axis_name="c", subcore_axis_name="s", num_cores=NC, num_subcores=NS)


def _deg_body(dst_hbm, out_hbm, idx_v, ones_v, zero_v, acc_sh):
    c = lax.axis_index("c")
    s = lax.axis_index("s")
    tid = c * NS + s
    for j in range(CH // 16):
        ones_v[pl.ds(16 * j, 16)] = jnp.ones((16,), jnp.float32)
    for j in range(RPT // 16):
        zero_v[pl.ds(16 * j, 16)] = jnp.zeros((16,), jnp.float32)
    pltpu.sync_copy(zero_v, acc_sh.at[pl.ds(s * RPT, RPT)])
    plsc.subcore_barrier()
    base = tid * EPT

    def body(i, carry):
        pltpu.sync_copy(dst_hbm.at[pl.ds(base + i * CH, CH)], idx_v)
        pltpu.sync_copy(ones_v, acc_sh.at[idx_v], add=True)
        return carry

    lax.fori_loop(0, NCHUNK, body, 0)
    plsc.subcore_barrier()
    pltpu.sync_copy(acc_sh.at[pl.ds(s * RPT, RPT)],
                    out_hbm.at[c, pl.ds(s * RPT, RPT)])


def _sc_degree(dst_pad):
    k = pl.kernel(
        _deg_body,
        out_type=jax.ShapeDtypeStruct((NC, NPAD), jnp.float32),
        mesh=_mesh(),
        scratch_types=[
            pltpu.VMEM((CH,), jnp.int32),
            pltpu.VMEM((CH,), jnp.float32),
            pltpu.VMEM((RPT,), jnp.float32),
            pltpu.VMEM_SHARED((NPAD,), jnp.float32),
        ],
    )
    return k(dst_pad)


def _scat_body(src_hbm, dst_hbm, g_hbm, out_hbm, sidx0, sidx1, didx0, didx1,
               rows0, rows1, acc_sh, sem0, sem1):
    c = lax.axis_index("c")
    s = lax.axis_index("s")
    tid = c * NS + s
    bufs = ((sidx0, didx0, rows0, sem0), (sidx1, didx1, rows1, sem1))

    def zbody(r, carry):
        for j in range(D // 16):
            rows0[r, pl.ds(16 * j, 16)] = jnp.zeros((16,), jnp.float32)
        return carry

    lax.fori_loop(0, CH, zbody, 0)
    for k in range(RPT // CH):
        pltpu.sync_copy(rows0, acc_sh.at[pl.ds(s * RPT + k * CH, CH)])
    plsc.subcore_barrier()
    base = tid * EPT

    for b, (sidx, didx, rows, sem) in enumerate(bufs):
        pltpu.sync_copy(src_hbm.at[pl.ds(base + b * CH, CH)], sidx)
        pltpu.sync_copy(dst_hbm.at[pl.ds(base + b * CH, CH)], didx)
        pltpu.async_copy(g_hbm.at[sidx], rows, sem)

    ngrp = NCHUNK // 2

    def ebody(p, carry):
        for b, (sidx, didx, rows, sem) in enumerate(bufs):
            pltpu.make_async_copy(g_hbm.at[sidx], rows, sem).wait()
            pltpu.sync_copy(rows, acc_sh.at[didx], add=True)

            @pl.when(p < ngrp - 1)
            def _refire():
                off = base + ((p + 1) * 2 + b) * CH
                pltpu.sync_copy(src_hbm.at[pl.ds(off, CH)], sidx)
                pltpu.sync_copy(dst_hbm.at[pl.ds(off, CH)], didx)
                pltpu.async_copy(g_hbm.at[sidx], rows, sem)
        return carry

    lax.fori_loop(0, ngrp, ebody, 0)
    plsc.subcore_barrier()
    pltpu.sync_copy(acc_sh.at[pl.ds(s * RPT, RPT)],
                    out_hbm.at[c, pl.ds(s * RPT, RPT)])


def _sc_scatter(src_pad, dst_pad, g):
    k = pl.kernel(
        _scat_body,
        out_type=jax.ShapeDtypeStruct((NC, NPAD, D), jnp.float32),
        mesh=_mesh(),
        scratch_types=[
            pltpu.VMEM((CH,), jnp.int32),
            pltpu.VMEM((CH,), jnp.int32),
            pltpu.VMEM((CH,), jnp.int32),
            pltpu.VMEM((CH,), jnp.int32),
            pltpu.VMEM((CH, D), jnp.float32),
            pltpu.VMEM((CH, D), jnp.float32),
            pltpu.VMEM_SHARED((NPAD, D), jnp.float32),
            pltpu.SemaphoreType.DMA,
            pltpu.SemaphoreType.DMA,
        ],
    )
    return k(src_pad, dst_pad, g)


def _mm1_body(x_ref, w_ref, deg_ref, g_ref, dis_ref):
    i = pl.program_id(0)
    deg = deg_ref[0] + deg_ref[1] + 1.0
    row = i * BLK + lax.broadcasted_iota(jnp.int32, (BLK, 1), 0)
    dis = jnp.where(row < N, lax.rsqrt(deg), 0.0)
    dis_ref[...] = dis
    t = jnp.dot(x_ref[...], w_ref[...], preferred_element_type=jnp.float32)
    g_ref[...] = t * dis


def _tc_mm1(x_pad, w, deg_col):
    return pl.pallas_call(
        _mm1_body,
        grid=(NBLK,),
        in_specs=[
            pl.BlockSpec((BLK, D), lambda i: (i, 0)),
            pl.BlockSpec((D, D), lambda i: (0, 0)),
            pl.BlockSpec((NC, BLK, 1), lambda i: (0, i, 0)),
        ],
        out_specs=[
            pl.BlockSpec((BLK, D), lambda i: (i, 0)),
            pl.BlockSpec((BLK, 1), lambda i: (i, 0)),
        ],
        out_shape=[
            jax.ShapeDtypeStruct((NPAD, D), jnp.float32),
            jax.ShapeDtypeStruct((NPAD, 1), jnp.float32),
        ],
    )(x_pad, w, deg_col)


def _ep_body(s_ref, g_ref, dis_ref, b_ref, w_ref, out_ref):
    dis = dis_ref[...]
    h = dis * (s_ref[0] + s_ref[1] + g_ref[...]) + b_ref[...]
    h = jnp.maximum(h, 0.0)
    out_ref[...] = jnp.dot(h, w_ref[...], preferred_element_type=jnp.float32) * dis


def _tc_epilogue(scat, g, dis_col, b, w):
    return pl.pallas_call(
        _ep_body,
        grid=(NBLK,),
        in_specs=[
            pl.BlockSpec((NC, BLK, D), lambda i: (0, i, 0)),
            pl.BlockSpec((BLK, D), lambda i: (i, 0)),
            pl.BlockSpec((BLK, 1), lambda i: (i, 0)),
            pl.BlockSpec((1, D), lambda i: (0, 0)),
            pl.BlockSpec((D, D), lambda i: (0, 0)),
        ],
        out_specs=pl.BlockSpec((BLK, D), lambda i: (i, 0)),
        out_shape=jax.ShapeDtypeStruct((NPAD, D), jnp.float32),
    )(scat, g, dis_col, b, w)


def _fin_body(s_ref, g_ref, dis_ref, b_ref, out_ref):
    out_ref[...] = (dis_ref[...] * (s_ref[0] + s_ref[1] + g_ref[...])
                    + b_ref[...])


def _tc_final(scat, g, dis_col, b):
    return pl.pallas_call(
        _fin_body,
        grid=(NBLK,),
        in_specs=[
            pl.BlockSpec((NC, BLK, D), lambda i: (0, i, 0)),
            pl.BlockSpec((BLK, D), lambda i: (i, 0)),
            pl.BlockSpec((BLK, 1), lambda i: (i, 0)),
            pl.BlockSpec((1, D), lambda i: (0, 0)),
        ],
        out_specs=pl.BlockSpec((BLK, D), lambda i: (i, 0)),
        out_shape=jax.ShapeDtypeStruct((NPAD, D), jnp.float32),
    )(scat, g, dis_col, b)


def kernel(x, edge_index, W1, b1, W2, b2, W3, b3):
    src = edge_index[0].astype(jnp.int32)
    dst = edge_index[1].astype(jnp.int32)
    npad_e = EPAD - E
    src_pad = jnp.concatenate([src, jnp.full((npad_e,), N, jnp.int32)])
    dst_pad = jnp.concatenate([dst, jnp.full((npad_e,), N + 8, jnp.int32)])
    x_pad = jnp.concatenate([x, jnp.zeros((NPAD - N, D), jnp.float32)])
    b1c = b1.reshape(1, D)
    b2c = b2.reshape(1, D)
    b3c = b3.reshape(1, D)

    deg = _sc_degree(dst_pad)
    deg_col = deg.reshape(NC, NPAD, 1)
    g1, dis_col = _tc_mm1(x_pad, W1, deg_col)
    s1 = _sc_scatter(src_pad, dst_pad, g1)
    g2 = _tc_epilogue(s1, g1, dis_col, b1c, W2)
    s2 = _sc_scatter(src_pad, dst_pad, g2)
    g3 = _tc_epilogue(s2, g2, dis_col, b2c, W3)
    s3 = _sc_scatter(src_pad, dst_pad, g3)
    out = _tc_final(s3, g3, dis_col, b3c)
    return out[:N]


# speedup vs baseline: 1.4808x; 1.2338x over previous
import jax
import jax.numpy as jnp
from jax import lax
from jax.experimental import pallas as pl
from jax.experimental.pallas import tpu as pltpu
from jax.experimental.pallas import tpu_sc as plsc

N = 10000
E = 320000
D = 128

NC, NS = 2, 16
NW = NC * NS
RPT = 640
NPAD = NS * RPT
CH = 128
NCHUNK = 79
EPT = NCHUNK * CH
EPAD = NW * EPT

BLK = 640
NBLK = NPAD // BLK

_mesh = lambda: plsc.VectorSubcoreMesh(
    core_axis_name="c", subcore_axis_name="s", num_cores=NC, num_subcores=NS)


def _deg_body(dst_hbm, out_hbm, idx_v, ones_v, zero_v, acc_sh):
    c = lax.axis_index("c")
    s = lax.axis_index("s")
    tid = c * NS + s
    for j in range(CH // 16):
        ones_v[pl.ds(16 * j, 16)] = jnp.ones((16,), jnp.float32)
    for j in range(RPT // 16):
        zero_v[pl.ds(16 * j, 16)] = jnp.zeros((16,), jnp.float32)
    pltpu.sync_copy(zero_v, acc_sh.at[pl.ds(s * RPT, RPT)])
    plsc.subcore_barrier()
    base = tid * EPT

    def body(i, carry):
        pltpu.sync_copy(dst_hbm.at[pl.ds(base + i * CH, CH)], idx_v)
        pltpu.sync_copy(ones_v, acc_sh.at[idx_v], add=True)
        return carry

    lax.fori_loop(0, NCHUNK, body, 0)
    plsc.subcore_barrier()
    pltpu.sync_copy(acc_sh.at[pl.ds(s * RPT, RPT)],
                    out_hbm.at[c, pl.ds(s * RPT, RPT)])


def _sc_degree(dst_pad):
    k = pl.kernel(
        _deg_body,
        out_type=jax.ShapeDtypeStruct((NC, NPAD), jnp.float32),
        mesh=_mesh(),
        scratch_types=[
            pltpu.VMEM((CH,), jnp.int32),
            pltpu.VMEM((CH,), jnp.float32),
            pltpu.VMEM((RPT,), jnp.float32),
            pltpu.VMEM_SHARED((NPAD,), jnp.float32),
        ],
    )
    return k(dst_pad)


def _scat_body(src_hbm, dst_hbm, g_hbm, out_hbm, sidx, didx, rows, acc_sh, sem):
    c = lax.axis_index("c")
    s = lax.axis_index("s")
    tid = c * NS + s

    def zbody(r, carry):
        for j in range(D // 16):
            rows[r, pl.ds(16 * j, 16)] = jnp.zeros((16,), jnp.float32)
        return carry

    lax.fori_loop(0, CH, zbody, 0)
    for k in range(RPT // CH):
        pltpu.sync_copy(rows, acc_sh.at[pl.ds(s * RPT + k * CH, CH)])
    plsc.subcore_barrier()
    base = tid * EPT

    def ebody(i, carry):
        off = base + i * CH
        pltpu.sync_copy(src_hbm.at[pl.ds(off, CH)], sidx)
        pltpu.sync_copy(dst_hbm.at[pl.ds(off, CH)], didx)
        pltpu.async_copy(g_hbm.at[sidx], rows, sem).wait()
        pltpu.sync_copy(rows, acc_sh.at[didx], add=True)
        return carry

    lax.fori_loop(0, NCHUNK, ebody, 0)
    plsc.subcore_barrier()
    pltpu.sync_copy(acc_sh.at[pl.ds(s * RPT, RPT)],
                    out_hbm.at[c, pl.ds(s * RPT, RPT)])


def _sc_scatter(src_pad, dst_pad, g):
    k = pl.kernel(
        _scat_body,
        out_type=jax.ShapeDtypeStruct((NC, NPAD, D), jnp.float32),
        mesh=_mesh(),
        scratch_types=[
            pltpu.VMEM((CH,), jnp.int32),
            pltpu.VMEM((CH,), jnp.int32),
            pltpu.VMEM((CH, D), jnp.float32),
            pltpu.VMEM_SHARED((NPAD, D), jnp.float32),
            pltpu.SemaphoreType.DMA,
        ],
    )
    return k(src_pad, dst_pad, g)


def _mm1_body(x_ref, w_ref, deg_ref, g_ref, dis_ref):
    i = pl.program_id(0)
    deg = deg_ref[0] + deg_ref[1] + 1.0
    row = i * BLK + lax.broadcasted_iota(jnp.int32, (BLK, 1), 0)
    dis = jnp.where(row < N, lax.rsqrt(deg), 0.0)
    dis_ref[...] = dis
    t = jnp.dot(x_ref[...], w_ref[...], preferred_element_type=jnp.float32)
    g_ref[...] = t * dis


def _tc_mm1(x_pad, w, deg_col):
    return pl.pallas_call(
        _mm1_body,
        grid=(NBLK,),
        in_specs=[
            pl.BlockSpec((BLK, D), lambda i: (i, 0)),
            pl.BlockSpec((D, D), lambda i: (0, 0)),
            pl.BlockSpec((NC, BLK, 1), lambda i: (0, i, 0)),
        ],
        out_specs=[
            pl.BlockSpec((BLK, D), lambda i: (i, 0)),
            pl.BlockSpec((BLK, 1), lambda i: (i, 0)),
        ],
        out_shape=[
            jax.ShapeDtypeStruct((NPAD, D), jnp.float32),
            jax.ShapeDtypeStruct((NPAD, 1), jnp.float32),
        ],
    )(x_pad, w, deg_col)


def _ep_body(s_ref, g_ref, dis_ref, b_ref, w_ref, out_ref):
    dis = dis_ref[...]
    h = dis * (s_ref[0] + s_ref[1] + g_ref[...]) + b_ref[...]
    h = jnp.maximum(h, 0.0)
    out_ref[...] = jnp.dot(h, w_ref[...], preferred_element_type=jnp.float32) * dis


def _tc_epilogue(scat, g, dis_col, b, w):
    return pl.pallas_call(
        _ep_body,
        grid=(NBLK,),
        in_specs=[
            pl.BlockSpec((NC, BLK, D), lambda i: (0, i, 0)),
            pl.BlockSpec((BLK, D), lambda i: (i, 0)),
            pl.BlockSpec((BLK, 1), lambda i: (i, 0)),
            pl.BlockSpec((1, D), lambda i: (0, 0)),
            pl.BlockSpec((D, D), lambda i: (0, 0)),
        ],
        out_specs=pl.BlockSpec((BLK, D), lambda i: (i, 0)),
        out_shape=jax.ShapeDtypeStruct((NPAD, D), jnp.float32),
    )(scat, g, dis_col, b, w)


def _fin_body(s_ref, g_ref, dis_ref, b_ref, out_ref):
    out_ref[...] = (dis_ref[...] * (s_ref[0] + s_ref[1] + g_ref[...])
                    + b_ref[...])


def _tc_final(scat, g, dis_col, b):
    return pl.pallas_call(
        _fin_body,
        grid=(NBLK,),
        in_specs=[
            pl.BlockSpec((NC, BLK, D), lambda i: (0, i, 0)),
            pl.BlockSpec((BLK, D), lambda i: (i, 0)),
            pl.BlockSpec((BLK, 1), lambda i: (i, 0)),
            pl.BlockSpec((1, D), lambda i: (0, 0)),
        ],
        out_specs=pl.BlockSpec((BLK, D), lambda i: (i, 0)),
        out_shape=jax.ShapeDtypeStruct((NPAD, D), jnp.float32),
    )(scat, g, dis_col, b)


def kernel(x, edge_index, W1, b1, W2, b2, W3, b3):
    src = edge_index[0].astype(jnp.int32)
    dst = edge_index[1].astype(jnp.int32)
    npad_e = EPAD - E
    src_pad = jnp.concatenate([src, jnp.full((npad_e,), N, jnp.int32)])
    dst_pad = jnp.concatenate([dst, jnp.full((npad_e,), N + 8, jnp.int32)])
    x_pad = jnp.concatenate([x, jnp.zeros((NPAD - N, D), jnp.float32)])
    b1c = b1.reshape(1, D)
    b2c = b2.reshape(1, D)
    b3c = b3.reshape(1, D)

    deg = _sc_degree(dst_pad)
    deg_col = deg.reshape(NC, NPAD, 1)
    g1, dis_col = _tc_mm1(x_pad, W1, deg_col)
    s1 = _sc_scatter(src_pad, dst_pad, g1)
    g2 = _tc_epilogue(s1, g1, dis_col, b1c, W2)
    s2 = _sc_scatter(src_pad, dst_pad, g2)
    g3 = _tc_epilogue(s2, g2, dis_col, b2c, W3)
    s3 = _sc_scatter(src_pad, dst_pad, g3)
    out = _tc_final(s3, g3, dis_col, b3c)
    return out[:N]


# speedup vs baseline: 2.1652x; 1.4621x over previous
import jax
import jax.numpy as jnp
from jax import lax
from jax.experimental import pallas as pl
from jax.experimental.pallas import tpu as pltpu
from jax.experimental.pallas import tpu_sc as plsc

N = 10000
E = 320000
D = 128

NC, NS = 2, 16
NW = NC * NS
RPT = 640
NPAD = NS * RPT
CH = 128
NCHUNK = 79
EPT = NCHUNK * CH
EPAD = NW * EPT

BLK = 640
NBLK = NPAD // BLK

_mesh = lambda: plsc.VectorSubcoreMesh(
    core_axis_name="c", subcore_axis_name="s", num_cores=NC, num_subcores=NS)


def _deg_body(dst_hbm, out_hbm, idx_v, ones_v, zero_v, acc_sh):
    c = lax.axis_index("c")
    s = lax.axis_index("s")
    tid = c * NS + s
    for j in range(CH // 16):
        ones_v[pl.ds(16 * j, 16)] = jnp.ones((16,), jnp.float32)
    for j in range(RPT // 16):
        zero_v[pl.ds(16 * j, 16)] = jnp.zeros((16,), jnp.float32)
    pltpu.sync_copy(zero_v, acc_sh.at[pl.ds(s * RPT, RPT)])
    plsc.subcore_barrier()
    base = tid * EPT

    def body(i, carry):
        pltpu.sync_copy(dst_hbm.at[pl.ds(base + i * CH, CH)], idx_v)
        pltpu.sync_copy(ones_v, acc_sh.at[idx_v], add=True)
        return carry

    lax.fori_loop(0, NCHUNK, body, 0)
    plsc.subcore_barrier()
    pltpu.sync_copy(acc_sh.at[pl.ds(s * RPT, RPT)],
                    out_hbm.at[c, pl.ds(s * RPT, RPT)])


def _sc_degree(dst_pad):
    k = pl.kernel(
        _deg_body,
        out_type=jax.ShapeDtypeStruct((NC, NPAD), jnp.float32),
        mesh=_mesh(),
        scratch_types=[
            pltpu.VMEM((CH,), jnp.int32),
            pltpu.VMEM((CH,), jnp.float32),
            pltpu.VMEM((RPT,), jnp.float32),
            pltpu.VMEM_SHARED((NPAD,), jnp.float32),
        ],
    )
    return k(dst_pad)


def _scat_body(src_hbm, dst_hbm, g_hbm, out_hbm, sidx, didx, rows, acc_sh, sem):
    c = lax.axis_index("c")
    s = lax.axis_index("s")
    tid = c * NS + s

    def zbody(r, carry):
        for j in range(D // 16):
            rows[r, pl.ds(16 * j, 16)] = jnp.zeros((16,), jnp.float32)
        return carry

    lax.fori_loop(0, CH, zbody, 0)
    for k in range(RPT // CH):
        pltpu.sync_copy(rows, acc_sh.at[pl.ds(s * RPT + k * CH, CH)])
    plsc.subcore_barrier()
    base = tid * EPT

    def ebody(i, carry):
        off = base + i * CH
        pltpu.sync_copy(src_hbm.at[pl.ds(off, CH)], sidx)
        pltpu.sync_copy(dst_hbm.at[pl.ds(off, CH)], didx)
        pltpu.async_copy(g_hbm.at[sidx], rows, sem).wait()
        pltpu.sync_copy(rows, acc_sh.at[didx], add=True)
        return carry

    lax.fori_loop(0, NCHUNK, ebody, 0)
    plsc.subcore_barrier()
    pltpu.sync_copy(acc_sh.at[pl.ds(s * RPT, RPT)],
                    out_hbm.at[c, pl.ds(s * RPT, RPT)])


def _sc_scatter(src_pad, dst_pad, g):
    k = pl.kernel(
        _scat_body,
        out_type=jax.ShapeDtypeStruct((NC, NPAD, D), jnp.float32),
        mesh=_mesh(),
        scratch_types=[
            pltpu.VMEM((CH,), jnp.int32),
            pltpu.VMEM((CH,), jnp.int32),
            pltpu.VMEM((CH, D), jnp.float32),
            pltpu.VMEM_SHARED((NPAD, D), jnp.float32),
            pltpu.SemaphoreType.DMA,
        ],
    )
    return k(src_pad, dst_pad, g)


def _mm1_body(x_ref, w_ref, deg_ref, g_ref, dis_ref):
    i = pl.program_id(0)
    deg = deg_ref[0] + deg_ref[1] + 1.0
    row = i * BLK + lax.broadcasted_iota(jnp.int32, (BLK, 1), 0)
    dis = jnp.where(row < N, lax.rsqrt(deg), 0.0)
    dis_ref[...] = dis
    t = jnp.dot(x_ref[...], w_ref[...], preferred_element_type=jnp.float32)
    g_ref[...] = t * dis


def _tc_mm1(x_pad, w, deg_col):
    return pl.pallas_call(
        _mm1_body,
        grid=(NBLK,),
        in_specs=[
            pl.BlockSpec((BLK, D), lambda i: (i, 0)),
            pl.BlockSpec((D, D), lambda i: (0, 0)),
            pl.BlockSpec((NC, BLK, 1), lambda i: (0, i, 0)),
        ],
        out_specs=[
            pl.BlockSpec((BLK, D), lambda i: (i, 0)),
            pl.BlockSpec((BLK, 1), lambda i: (i, 0)),
        ],
        out_shape=[
            jax.ShapeDtypeStruct((NPAD, D), jnp.float32),
            jax.ShapeDtypeStruct((NPAD, 1), jnp.float32),
        ],
    )(x_pad, w, deg_col)


def _ep_body(s_ref, g_ref, dis_ref, b_ref, w_ref, out_ref):
    dis = dis_ref[...]
    h = dis * (s_ref[0] + s_ref[1] + g_ref[...]) + b_ref[...]
    h = jnp.maximum(h, 0.0)
    out_ref[...] = jnp.dot(h, w_ref[...], preferred_element_type=jnp.float32) * dis


def _tc_epilogue(scat, g, dis_col, b, w):
    return pl.pallas_call(
        _ep_body,
        grid=(NBLK,),
        in_specs=[
            pl.BlockSpec((NC, BLK, D), lambda i: (0, i, 0)),
            pl.BlockSpec((BLK, D), lambda i: (i, 0)),
            pl.BlockSpec((BLK, 1), lambda i: (i, 0)),
            pl.BlockSpec((1, D), lambda i: (0, 0)),
            pl.BlockSpec((D, D), lambda i: (0, 0)),
        ],
        out_specs=pl.BlockSpec((BLK, D), lambda i: (i, 0)),
        out_shape=jax.ShapeDtypeStruct((NPAD, D), jnp.float32),
    )(scat, g, dis_col, b, w)


def _fin_body(s_ref, g_ref, dis_ref, b_ref, out_ref):
    out_ref[...] = (dis_ref[...] * (s_ref[0] + s_ref[1] + g_ref[...])
                    + b_ref[...])


def _tc_final(scat, g, dis_col, b):
    return pl.pallas_call(
        _fin_body,
        grid=(NBLK,),
        in_specs=[
            pl.BlockSpec((NC, BLK, D), lambda i: (0, i, 0)),
            pl.BlockSpec((BLK, D), lambda i: (i, 0)),
            pl.BlockSpec((BLK, 1), lambda i: (i, 0)),
            pl.BlockSpec((1, D), lambda i: (0, 0)),
        ],
        out_specs=pl.BlockSpec((BLK, D), lambda i: (i, 0)),
        out_shape=jax.ShapeDtypeStruct((NPAD, D), jnp.float32),
    )(scat, g, dis_col, b)


def kernel(x, edge_index, W1, b1, W2, b2, W3, b3):
    src = edge_index[0].astype(jnp.int32)
    dst = edge_index[1].astype(jnp.int32)
    npad_e = EPAD - E
    spread = jnp.arange(npad_e, dtype=jnp.int32) % (NPAD - N)
    src_pad = jnp.concatenate([src, N + spread])
    dst_pad = jnp.concatenate([dst, N + (NPAD - N - 1) - spread])
    x_pad = jnp.concatenate([x, jnp.zeros((NPAD - N, D), jnp.float32)])
    b1c = b1.reshape(1, D)
    b2c = b2.reshape(1, D)
    b3c = b3.reshape(1, D)

    deg = _sc_degree(dst_pad)
    deg_col = deg.reshape(NC, NPAD, 1)
    g1, dis_col = _tc_mm1(x_pad, W1, deg_col)
    s1 = _sc_scatter(src_pad, dst_pad, g1)
    g2 = _tc_epilogue(s1, g1, dis_col, b1c, W2)
    s2 = _sc_scatter(src_pad, dst_pad, g2)
    g3 = _tc_epilogue(s2, g2, dis_col, b2c, W3)
    s3 = _sc_scatter(src_pad, dst_pad, g3)
    out = _tc_final(s3, g3, dis_col, b3c)
    return out[:N]


# speedup vs baseline: 3.2020x; 1.4788x over previous
import jax
import jax.numpy as jnp
from jax import lax
from jax.experimental import pallas as pl
from jax.experimental.pallas import tpu as pltpu
from jax.experimental.pallas import tpu_sc as plsc

N = 10000
E = 320000
D = 128

NC, NS = 2, 16
NW = NC * NS
RPT = 640
NPAD = NS * RPT
CH = 128
NCHUNK = 80
EPT = NCHUNK * CH
EPAD = NW * EPT

BLK = 640
NBLK = NPAD // BLK

_mesh = lambda: plsc.VectorSubcoreMesh(
    core_axis_name="c", subcore_axis_name="s", num_cores=NC, num_subcores=NS)


def _deg_body(dst_hbm, out_hbm, idx_v, ones_v, zero_v, acc_sh):
    c = lax.axis_index("c")
    s = lax.axis_index("s")
    tid = c * NS + s
    for j in range(CH // 16):
        ones_v[pl.ds(16 * j, 16)] = jnp.ones((16,), jnp.float32)
    for j in range(RPT // 16):
        zero_v[pl.ds(16 * j, 16)] = jnp.zeros((16,), jnp.float32)
    pltpu.sync_copy(zero_v, acc_sh.at[pl.ds(s * RPT, RPT)])
    plsc.subcore_barrier()
    base = tid * EPT

    def body(i, carry):
        pltpu.sync_copy(dst_hbm.at[pl.ds(base + i * CH, CH)], idx_v)
        pltpu.sync_copy(ones_v, acc_sh.at[idx_v], add=True)
        return carry

    lax.fori_loop(0, NCHUNK, body, 0)
    plsc.subcore_barrier()
    pltpu.sync_copy(acc_sh.at[pl.ds(s * RPT, RPT)],
                    out_hbm.at[c, pl.ds(s * RPT, RPT)])


def _sc_degree(dst_pad):
    k = pl.kernel(
        _deg_body,
        out_type=jax.ShapeDtypeStruct((NC, NPAD), jnp.float32),
        mesh=_mesh(),
        scratch_types=[
            pltpu.VMEM((CH,), jnp.int32),
            pltpu.VMEM((CH,), jnp.float32),
            pltpu.VMEM((RPT,), jnp.float32),
            pltpu.VMEM_SHARED((NPAD,), jnp.float32),
        ],
    )
    return k(dst_pad)


def _scat_body(src_hbm, dst_hbm, g_hbm, out_hbm, sidx0, sidx1, didx0, didx1,
               rows0, rows1, acc_sh, sem0, sem1):
    c = lax.axis_index("c")
    s = lax.axis_index("s")
    tid = c * NS + s
    bufs = ((sidx0, didx0, rows0, sem0), (sidx1, didx1, rows1, sem1))

    def zbody(r, carry):
        for j in range(D // 16):
            rows0[r, pl.ds(16 * j, 16)] = jnp.zeros((16,), jnp.float32)
        return carry

    lax.fori_loop(0, CH, zbody, 0)
    for k in range(RPT // CH):
        pltpu.sync_copy(rows0, acc_sh.at[pl.ds(s * RPT + k * CH, CH)])
    plsc.subcore_barrier()
    base = tid * EPT

    for b, (sidx, didx, rows, sem) in enumerate(bufs):
        pltpu.sync_copy(src_hbm.at[pl.ds(base + b * CH, CH)], sidx)
        pltpu.sync_copy(dst_hbm.at[pl.ds(base + b * CH, CH)], didx)
        pltpu.async_copy(g_hbm.at[sidx], rows, sem)

    def ebody(p, carry):
        for b, (sidx, didx, rows, sem) in enumerate(bufs):
            pltpu.make_async_copy(g_hbm.at[sidx], rows, sem).wait()
            pltpu.sync_copy(rows, acc_sh.at[didx], add=True)
            off = base + ((p + 1) * 2 + b) * CH
            pltpu.sync_copy(src_hbm.at[pl.ds(off, CH)], sidx)
            pltpu.sync_copy(dst_hbm.at[pl.ds(off, CH)], didx)
            pltpu.async_copy(g_hbm.at[sidx], rows, sem)
        return carry

    lax.fori_loop(0, NCHUNK // 2 - 1, ebody, 0)
    for b, (sidx, didx, rows, sem) in enumerate(bufs):
        pltpu.make_async_copy(g_hbm.at[sidx], rows, sem).wait()
        pltpu.sync_copy(rows, acc_sh.at[didx], add=True)
    plsc.subcore_barrier()
    pltpu.sync_copy(acc_sh.at[pl.ds(s * RPT, RPT)],
                    out_hbm.at[c, pl.ds(s * RPT, RPT)])


def _sc_scatter(src_pad, dst_pad, g):
    k = pl.kernel(
        _scat_body,
        out_type=jax.ShapeDtypeStruct((NC, NPAD, D), jnp.float32),
        mesh=_mesh(),
        scratch_types=[
            pltpu.VMEM((CH,), jnp.int32),
            pltpu.VMEM((CH,), jnp.int32),
            pltpu.VMEM((CH,), jnp.int32),
            pltpu.VMEM((CH,), jnp.int32),
            pltpu.VMEM((CH, D), jnp.float32),
            pltpu.VMEM((CH, D), jnp.float32),
            pltpu.VMEM_SHARED((NPAD, D), jnp.float32),
            pltpu.SemaphoreType.DMA,
            pltpu.SemaphoreType.DMA,
        ],
    )
    return k(src_pad, dst_pad, g)


def _mm1_body(x_ref, w_ref, deg_ref, g_ref, dis_ref):
    i = pl.program_id(0)
    deg = deg_ref[0] + deg_ref[1] + 1.0
    row = i * BLK + lax.broadcasted_iota(jnp.int32, (BLK, 1), 0)
    dis = jnp.where(row < N, lax.rsqrt(deg), 0.0)
    dis_ref[...] = dis
    t = jnp.dot(x_ref[...], w_ref[...], preferred_element_type=jnp.float32)
    g_ref[...] = t * dis


def _tc_mm1(x_pad, w, deg_col):
    return pl.pallas_call(
        _mm1_body,
        grid=(NBLK,),
        in_specs=[
            pl.BlockSpec((BLK, D), lambda i: (i, 0)),
            pl.BlockSpec((D, D), lambda i: (0, 0)),
            pl.BlockSpec((NC, BLK, 1), lambda i: (0, i, 0)),
        ],
        out_specs=[
            pl.BlockSpec((BLK, D), lambda i: (i, 0)),
            pl.BlockSpec((BLK, 1), lambda i: (i, 0)),
        ],
        out_shape=[
            jax.ShapeDtypeStruct((NPAD, D), jnp.float32),
            jax.ShapeDtypeStruct((NPAD, 1), jnp.float32),
        ],
    )(x_pad, w, deg_col)


def _ep_body(s_ref, g_ref, dis_ref, b_ref, w_ref, out_ref):
    dis = dis_ref[...]
    h = dis * (s_ref[0] + s_ref[1] + g_ref[...]) + b_ref[...]
    h = jnp.maximum(h, 0.0)
    out_ref[...] = jnp.dot(h, w_ref[...], preferred_element_type=jnp.float32) * dis


def _tc_epilogue(scat, g, dis_col, b, w):
    return pl.pallas_call(
        _ep_body,
        grid=(NBLK,),
        in_specs=[
            pl.BlockSpec((NC, BLK, D), lambda i: (0, i, 0)),
            pl.BlockSpec((BLK, D), lambda i: (i, 0)),
            pl.BlockSpec((BLK, 1), lambda i: (i, 0)),
            pl.BlockSpec((1, D), lambda i: (0, 0)),
            pl.BlockSpec((D, D), lambda i: (0, 0)),
        ],
        out_specs=pl.BlockSpec((BLK, D), lambda i: (i, 0)),
        out_shape=jax.ShapeDtypeStruct((NPAD, D), jnp.float32),
    )(scat, g, dis_col, b, w)


def _fin_body(s_ref, g_ref, dis_ref, b_ref, out_ref):
    out_ref[...] = (dis_ref[...] * (s_ref[0] + s_ref[1] + g_ref[...])
                    + b_ref[...])


def _tc_final(scat, g, dis_col, b):
    return pl.pallas_call(
        _fin_body,
        grid=(NBLK,),
        in_specs=[
            pl.BlockSpec((NC, BLK, D), lambda i: (0, i, 0)),
            pl.BlockSpec((BLK, D), lambda i: (i, 0)),
            pl.BlockSpec((BLK, 1), lambda i: (i, 0)),
            pl.BlockSpec((1, D), lambda i: (0, 0)),
        ],
        out_specs=pl.BlockSpec((BLK, D), lambda i: (i, 0)),
        out_shape=jax.ShapeDtypeStruct((NPAD, D), jnp.float32),
    )(scat, g, dis_col, b)


def kernel(x, edge_index, W1, b1, W2, b2, W3, b3):
    src = edge_index[0].astype(jnp.int32)
    dst = edge_index[1].astype(jnp.int32)
    npad_e = EPAD - E
    spread = jnp.arange(npad_e, dtype=jnp.int32) % (NPAD - N)
    src_pad = jnp.concatenate([src, N + spread])
    dst_pad = jnp.concatenate([dst, N + (NPAD - N - 1) - spread])
    x_pad = jnp.concatenate([x, jnp.zeros((NPAD - N, D), jnp.float32)])
    b1c = b1.reshape(1, D)
    b2c = b2.reshape(1, D)
    b3c = b3.reshape(1, D)

    deg = _sc_degree(dst_pad)
    deg_col = deg.reshape(NC, NPAD, 1)
    g1, dis_col = _tc_mm1(x_pad, W1, deg_col)
    s1 = _sc_scatter(src_pad, dst_pad, g1)
    g2 = _tc_epilogue(s1, g1, dis_col, b1c, W2)
    s2 = _sc_scatter(src_pad, dst_pad, g2)
    g3 = _tc_epilogue(s2, g2, dis_col, b2c, W3)
    s3 = _sc_scatter(src_pad, dst_pad, g3)
    out = _tc_final(s3, g3, dis_col, b3c)
    return out[:N]


# speedup vs baseline: 4.0798x; 1.2742x over previous
import jax
import jax.numpy as jnp
from jax import lax
from jax.experimental import pallas as pl
from jax.experimental.pallas import tpu as pltpu
from jax.experimental.pallas import tpu_sc as plsc

N = 10000
E = 320000
D = 128

NC, NS = 2, 16
NW = NC * NS
RPT = 640
NPAD = NS * RPT
CH = 128
NCHUNK = 80
EPT = NCHUNK * CH
EPAD = NW * EPT

BLK = 640
NBLK = NPAD // BLK

_mesh = lambda: plsc.VectorSubcoreMesh(
    core_axis_name="c", subcore_axis_name="s", num_cores=NC, num_subcores=NS)


def _deg_body(dst_hbm, out_hbm, idx_v, ones_v, zero_v, acc_sh):
    c = lax.axis_index("c")
    s = lax.axis_index("s")
    tid = c * NS + s
    for j in range(CH // 16):
        ones_v[pl.ds(16 * j, 16)] = jnp.ones((16,), jnp.float32)
    for j in range(RPT // 16):
        zero_v[pl.ds(16 * j, 16)] = jnp.zeros((16,), jnp.float32)
    pltpu.sync_copy(zero_v, acc_sh.at[pl.ds(s * RPT, RPT)])
    plsc.subcore_barrier()
    base = tid * EPT

    def body(i, carry):
        pltpu.sync_copy(dst_hbm.at[pl.ds(base + i * CH, CH)], idx_v)
        pltpu.sync_copy(ones_v, acc_sh.at[idx_v], add=True)
        return carry

    lax.fori_loop(0, NCHUNK, body, 0)
    plsc.subcore_barrier()
    pltpu.sync_copy(acc_sh.at[pl.ds(s * RPT, RPT)],
                    out_hbm.at[c, pl.ds(s * RPT, RPT)])


def _sc_degree(dst_pad):
    k = pl.kernel(
        _deg_body,
        out_type=jax.ShapeDtypeStruct((NC, NPAD), jnp.float32),
        mesh=_mesh(),
        scratch_types=[
            pltpu.VMEM((CH,), jnp.int32),
            pltpu.VMEM((CH,), jnp.float32),
            pltpu.VMEM((RPT,), jnp.float32),
            pltpu.VMEM_SHARED((NPAD,), jnp.float32),
        ],
    )
    return k(dst_pad)


def _scat_body(src_hbm, dst_hbm, g_hbm, out_hbm, sidx0, sidx1, didx0, didx1,
               rows0, rows1, acc_sh, gsem0, gsem1, ssem0, ssem1, dsem0, dsem1):
    c = lax.axis_index("c")
    s = lax.axis_index("s")
    tid = c * NS + s
    bufs = ((sidx0, didx0, rows0, gsem0, ssem0, dsem0),
            (sidx1, didx1, rows1, gsem1, ssem1, dsem1))

    def zbody(r, carry):
        for j in range(D // 16):
            rows0[r, pl.ds(16 * j, 16)] = jnp.zeros((16,), jnp.float32)
        return carry

    lax.fori_loop(0, CH, zbody, 0)
    for k in range(RPT // CH):
        pltpu.sync_copy(rows0, acc_sh.at[pl.ds(s * RPT + k * CH, CH)])
    plsc.subcore_barrier()
    base = tid * EPT

    for b, (sidx, didx, rows, gsem, ssem, dsem) in enumerate(bufs):
        pltpu.sync_copy(src_hbm.at[pl.ds(base + b * CH, CH)], sidx)
        pltpu.async_copy(dst_hbm.at[pl.ds(base + b * CH, CH)], didx, dsem)
        pltpu.async_copy(g_hbm.at[sidx], rows, gsem)

    def ebody(p, carry):
        for b, (sidx, didx, rows, gsem, ssem, dsem) in enumerate(bufs):
            off_c = base + (2 * p + b) * CH
            off_n = base + ((p + 1) * 2 + b) * CH
            pltpu.make_async_copy(g_hbm.at[sidx], rows, gsem).wait()
            pltpu.async_copy(src_hbm.at[pl.ds(off_n, CH)], sidx, ssem)
            pltpu.make_async_copy(
                dst_hbm.at[pl.ds(off_c, CH)], didx, dsem).wait()
            pltpu.sync_copy(rows, acc_sh.at[didx], add=True)
            pltpu.async_copy(dst_hbm.at[pl.ds(off_n, CH)], didx, dsem)
            pltpu.make_async_copy(
                src_hbm.at[pl.ds(off_n, CH)], sidx, ssem).wait()
            pltpu.async_copy(g_hbm.at[sidx], rows, gsem)
        return carry

    lax.fori_loop(0, NCHUNK // 2 - 1, ebody, 0)
    for b, (sidx, didx, rows, gsem, ssem, dsem) in enumerate(bufs):
        off_c = base + (NCHUNK - 2 + b) * CH
        pltpu.make_async_copy(g_hbm.at[sidx], rows, gsem).wait()
        pltpu.make_async_copy(dst_hbm.at[pl.ds(off_c, CH)], didx, dsem).wait()
        pltpu.sync_copy(rows, acc_sh.at[didx], add=True)
    plsc.subcore_barrier()
    pltpu.sync_copy(acc_sh.at[pl.ds(s * RPT, RPT)],
                    out_hbm.at[c, pl.ds(s * RPT, RPT)])


def _sc_scatter(src_pad, dst_pad, g):
    k = pl.kernel(
        _scat_body,
        out_type=jax.ShapeDtypeStruct((NC, NPAD, D), jnp.float32),
        mesh=_mesh(),
        scratch_types=[
            pltpu.VMEM((CH,), jnp.int32),
            pltpu.VMEM((CH,), jnp.int32),
            pltpu.VMEM((CH,), jnp.int32),
            pltpu.VMEM((CH,), jnp.int32),
            pltpu.VMEM((CH, D), jnp.float32),
            pltpu.VMEM((CH, D), jnp.float32),
            pltpu.VMEM_SHARED((NPAD, D), jnp.float32),
        ] + [pltpu.SemaphoreType.DMA] * 6,
    )
    return k(src_pad, dst_pad, g)


def _mm1_body(x_ref, w_ref, deg_ref, g_ref, dis_ref):
    i = pl.program_id(0)
    deg = deg_ref[0] + deg_ref[1] + 1.0
    row = i * BLK + lax.broadcasted_iota(jnp.int32, (BLK, 1), 0)
    dis = jnp.where(row < N, lax.rsqrt(deg), 0.0)
    dis_ref[...] = dis
    t = jnp.dot(x_ref[...], w_ref[...], preferred_element_type=jnp.float32)
    g_ref[...] = t * dis


def _tc_mm1(x_pad, w, deg_col):
    return pl.pallas_call(
        _mm1_body,
        grid=(NBLK,),
        in_specs=[
            pl.BlockSpec((BLK, D), lambda i: (i, 0)),
            pl.BlockSpec((D, D), lambda i: (0, 0)),
            pl.BlockSpec((NC, BLK, 1), lambda i: (0, i, 0)),
        ],
        out_specs=[
            pl.BlockSpec((BLK, D), lambda i: (i, 0)),
            pl.BlockSpec((BLK, 1), lambda i: (i, 0)),
        ],
        out_shape=[
            jax.ShapeDtypeStruct((NPAD, D), jnp.float32),
            jax.ShapeDtypeStruct((NPAD, 1), jnp.float32),
        ],
    )(x_pad, w, deg_col)


def _ep_body(s_ref, g_ref, dis_ref, b_ref, w_ref, out_ref):
    dis = dis_ref[...]
    h = dis * (s_ref[0] + s_ref[1] + g_ref[...]) + b_ref[...]
    h = jnp.maximum(h, 0.0)
    out_ref[...] = jnp.dot(h, w_ref[...], preferred_element_type=jnp.float32) * dis


def _tc_epilogue(scat, g, dis_col, b, w):
    return pl.pallas_call(
        _ep_body,
        grid=(NBLK,),
        in_specs=[
            pl.BlockSpec((NC, BLK, D), lambda i: (0, i, 0)),
            pl.BlockSpec((BLK, D), lambda i: (i, 0)),
            pl.BlockSpec((BLK, 1), lambda i: (i, 0)),
            pl.BlockSpec((1, D), lambda i: (0, 0)),
            pl.BlockSpec((D, D), lambda i: (0, 0)),
        ],
        out_specs=pl.BlockSpec((BLK, D), lambda i: (i, 0)),
        out_shape=jax.ShapeDtypeStruct((NPAD, D), jnp.float32),
    )(scat, g, dis_col, b, w)


def _fin_body(s_ref, g_ref, dis_ref, b_ref, out_ref):
    out_ref[...] = (dis_ref[...] * (s_ref[0] + s_ref[1] + g_ref[...])
                    + b_ref[...])


def _tc_final(scat, g, dis_col, b):
    return pl.pallas_call(
        _fin_body,
        grid=(NBLK,),
        in_specs=[
            pl.BlockSpec((NC, BLK, D), lambda i: (0, i, 0)),
            pl.BlockSpec((BLK, D), lambda i: (i, 0)),
            pl.BlockSpec((BLK, 1), lambda i: (i, 0)),
            pl.BlockSpec((1, D), lambda i: (0, 0)),
        ],
        out_specs=pl.BlockSpec((BLK, D), lambda i: (i, 0)),
        out_shape=jax.ShapeDtypeStruct((NPAD, D), jnp.float32),
    )(scat, g, dis_col, b)


def kernel(x, edge_index, W1, b1, W2, b2, W3, b3):
    src = edge_index[0].astype(jnp.int32)
    dst = edge_index[1].astype(jnp.int32)
    npad_e = EPAD - E
    spread = jnp.arange(npad_e, dtype=jnp.int32) % (NPAD - N)
    src_pad = jnp.concatenate([src, N + spread])
    dst_pad = jnp.concatenate([dst, N + (NPAD - N - 1) - spread])
    x_pad = jnp.concatenate([x, jnp.zeros((NPAD - N, D), jnp.float32)])
    b1c = b1.reshape(1, D)
    b2c = b2.reshape(1, D)
    b3c = b3.reshape(1, D)

    deg = _sc_degree(dst_pad)
    deg_col = deg.reshape(NC, NPAD, 1)
    g1, dis_col = _tc_mm1(x_pad, W1, deg_col)
    s1 = _sc_scatter(src_pad, dst_pad, g1)
    g2 = _tc_epilogue(s1, g1, dis_col, b1c, W2)
    s2 = _sc_scatter(src_pad, dst_pad, g2)
    g3 = _tc_epilogue(s2, g2, dis_col, b2c, W3)
    s3 = _sc_scatter(src_pad, dst_pad, g3)
    out = _tc_final(s3, g3, dis_col, b3c)
    return out[:N]
